# baseline (speedup 1.0000x reference)
import functools

import jax
import jax.numpy as jnp
from jax import lax
from jax.experimental import pallas as pl
from jax.experimental.pallas import tpu as pltpu
from jax.experimental.pallas import tpu_sc as plsc

_DEL = 16
_B = 16384
_NLANE = 1000000
_NTILE = 16
_BPT = _B // _NTILE
_NB = 64
_TAIL0 = (_NLANE // 128) * 128
_LAST_FULL_TC = _NLANE // 128 - 1


def _gather_sigmoid(idx, embeddings):
  mesh = plsc.VectorSubcoreMesh(core_axis_name="c", subcore_axis_name="s")

  @functools.partial(
      pl.kernel,
      out_type=jax.ShapeDtypeStruct((_DEL * _B,), jnp.float32),
      mesh=mesh,
      compiler_params=pltpu.CompilerParams(needs_layout_passes=False),
      scratch_types=[
          pltpu.VMEM((_BPT,), jnp.int32),
          pltpu.VMEM((_NB + 1, 8, 128), jnp.float32),
          pltpu.VMEM((8, _BPT), jnp.float32),
          pltpu.SemaphoreType.DMA,
      ],
  )
  def k(t3_hbm, tail_hbm, idx_hbm, out_hbm, idx_v, ring_v, vals_v, sem):
    c = lax.axis_index("c")
    s = lax.axis_index("s")
    base = s * _BPT
    pltpu.sync_copy(idx_hbm.at[pl.ds(base, _BPT)], idx_v)
    pltpu.sync_copy(tail_hbm.at[c], ring_v.at[_NB])

    js16 = lax.iota(jnp.int32, 16) & 7

    def group_vectors(g_start, slot_base):
      i16 = idx_v[pl.ds(g_start, 16)]
      tail = i16 >= _TAIL0
      tc = jnp.minimum(lax.shift_right_logical(i16, 7), _LAST_FULL_TC)
      off16 = tc * 128
      lane16 = jnp.where(tail, i16 - _TAIL0, i16 - off16)
      slotsel16 = jnp.where(tail, _NB, lax.iota(jnp.int32, 16) + slot_base)
      return off16, lane16, slotsel16

    def fire(off16, b, slot):
      off = pl.multiple_of(off16[b], 128)
      pltpu.async_copy(
          t3_hbm.at[c, :, pl.ds(off, 128)], ring_v.at[slot], sem
      )

    def extract(kpos, lane16, slotsel16, b):
      lane = lane16[b]
      slot_sel = slotsel16[b]
      x = plsc.load_gather(
          ring_v,
          [
              jnp.broadcast_to(slot_sel, (16,)),
              js16,
              jnp.broadcast_to(lane, (16,)),
          ],
      )
      y = 1.0 / (1.0 + jnp.exp(-x))
      plsc.store_scatter(
          vals_v, [js16, jnp.broadcast_to(kpos, (16,))], y
      )

    nh = 4
    gs = tuple(group_vectors(16 * h, 16 * h) for h in range(nh))
    for h in range(nh):
      for b in range(16):
        fire(gs[h][0], b, 16 * h + b)

    npairs = _BPT // (16 * nh)

    def body(g, carry):
      nxt0 = jnp.minimum((g + 1) * 16 * nh, _BPT - 16 * nh)
      nxts = tuple(group_vectors(nxt0 + 16 * h, 16 * h) for h in range(nh))
      more = g + 1 < npairs
      for half in range(nh):
        cur = carry[half]
        nxt = nxts[half]
        for b in range(16):
          kpos = g * 16 * nh + half * 16 + b
          slot = half * 16 + b
          pltpu.make_async_copy(
              t3_hbm.at[c, :, pl.ds(0, 128)], ring_v.at[slot], sem
          ).wait()
          extract(kpos, cur[1], cur[2], b)

          @pl.when(more)
          def _():
            fire(nxt[0], b, slot)

      return nxts

    lax.fori_loop(0, npairs, body, gs)

    for js in range(8):
      pltpu.sync_copy(
          vals_v.at[js],
          out_hbm.at[pl.ds((c * 8 + js) * _B + base, _BPT)],
      )

  tail = jnp.pad(embeddings[_TAIL0:], ((0, 128 - (_NLANE - _TAIL0)), (0, 0)))
  return k(embeddings.T.reshape(2, 8, _NLANE), tail.T.reshape(2, 8, 128), idx)


def kernel(idx, embeddings):
  out1d = _gather_sigmoid(idx, embeddings)
  return jnp.transpose(out1d.reshape(_DEL, 1, _B), (2, 0, 1))


# speedup vs baseline: 1.0145x; 1.0145x over previous
import functools

import jax
import jax.numpy as jnp
from jax import lax
from jax.experimental import pallas as pl
from jax.experimental.pallas import tpu as pltpu
from jax.experimental.pallas import tpu_sc as plsc

_DEL = 16
_B = 16384
_NLANE = 1000000
_NTILE = 16
_BPT = _B // _NTILE
_NB = 32
_TAIL0 = (_NLANE // 128) * 128
_LAST_FULL_TC = _NLANE // 128 - 1


def _gather_sigmoid(idx, embeddings):
  mesh = plsc.VectorSubcoreMesh(core_axis_name="c", subcore_axis_name="s")

  @functools.partial(
      pl.kernel,
      out_type=jax.ShapeDtypeStruct((_DEL * _B,), jnp.float32),
      mesh=mesh,
      compiler_params=pltpu.CompilerParams(needs_layout_passes=False),
      scratch_types=[
          pltpu.VMEM((_BPT,), jnp.int32),
          pltpu.VMEM((_NB + 1, 8, 128), jnp.float32),
          pltpu.VMEM((8, _BPT), jnp.float32),
          pltpu.SemaphoreType.DMA,
      ],
  )
  def k(t3_hbm, tail_hbm, idx_hbm, out_hbm, idx_v, ring_v, vals_v, sem):
    c = lax.axis_index("c")
    s = lax.axis_index("s")
    base = s * _BPT
    pltpu.sync_copy(idx_hbm.at[pl.ds(base, _BPT)], idx_v)
    pltpu.sync_copy(tail_hbm.at[c], ring_v.at[_NB])

    js16 = lax.iota(jnp.int32, 16) & 7

    def group_vectors(g_start, slot_base):
      i16 = idx_v[pl.ds(g_start, 16)]
      tail = i16 >= _TAIL0
      tc = jnp.minimum(lax.shift_right_logical(i16, 7), _LAST_FULL_TC)
      off16 = tc * 128
      lane16 = jnp.where(tail, i16 - _TAIL0, i16 - off16)
      slotsel16 = jnp.where(tail, _NB, lax.iota(jnp.int32, 16) + slot_base)
      return off16, lane16, slotsel16

    def fire(off16, b, slot):
      off = pl.multiple_of(off16[b], 128)
      pltpu.async_copy(
          t3_hbm.at[c, :, pl.ds(off, 128)], ring_v.at[slot], sem
      )

    def extract(kpos, lane16, slotsel16, b):
      lane = lane16[b]
      slot_sel = slotsel16[b]
      x = plsc.load_gather(
          ring_v,
          [
              jnp.broadcast_to(slot_sel, (16,)),
              js16,
              jnp.broadcast_to(lane, (16,)),
          ],
      )
      y = 1.0 / (1.0 + jnp.exp(-x))
      plsc.store_scatter(
          vals_v, [js16, jnp.broadcast_to(kpos, (16,))], y
      )

    nh = 2
    gs = tuple(group_vectors(16 * h, 16 * h) for h in range(nh))
    for h in range(nh):
      for b in range(16):
        fire(gs[h][0], b, 16 * h + b)

    npairs = _BPT // (16 * nh)

    def body(g, carry):
      nxt0 = jnp.minimum((g + 1) * 16 * nh, _BPT - 16 * nh)
      nxts = tuple(group_vectors(nxt0 + 16 * h, 16 * h) for h in range(nh))
      more = g + 1 < npairs
      for half in range(nh):
        cur = carry[half]
        nxt = nxts[half]
        for b in range(16):
          kpos = g * 16 * nh + half * 16 + b
          slot = half * 16 + b
          pltpu.make_async_copy(
              t3_hbm.at[c, :, pl.ds(0, 128)], ring_v.at[slot], sem
          ).wait()
          extract(kpos, cur[1], cur[2], b)

          @pl.when(more)
          def _():
            fire(nxt[0], b, slot)

      return nxts

    lax.fori_loop(0, npairs, body, gs)

    for js in range(8):
      pltpu.sync_copy(
          vals_v.at[js],
          out_hbm.at[pl.ds((c * 8 + js) * _B + base, _BPT)],
      )

  tail = jnp.pad(embeddings[_TAIL0:], ((0, 128 - (_NLANE - _TAIL0)), (0, 0)))
  return k(embeddings.T.reshape(2, 8, _NLANE), tail.T.reshape(2, 8, 128), idx)


def kernel(idx, embeddings):
  out1d = _gather_sigmoid(idx, embeddings)
  return jnp.transpose(out1d.reshape(_DEL, 1, _B), (2, 0, 1))
